# Initial kernel scaffold; baseline (speedup 1.0000x reference)
import jax
import jax.numpy as jnp
from jax.experimental import pallas as pl


def kernel(arm_loc, arm_conf, odm_loc, odm_conf, priors, targets):
    raise NotImplementedError("write your pallas kernel here")


# speedup vs baseline: 32.1883x; 32.1883x over previous
import jax
import jax.numpy as jnp
from jax.experimental import pallas as pl
from jax.experimental.pallas import tpu as pltpu

_NUM_CLASSES = 21
_POS_PRIOR_THRESHOLD = 0.01
_OVERLAP_THRESH = 0.5
_NEG_POS_RATIO = 3
_N, _P, _T = 16, 20000, 8
_LANES = 128
_PPAD = 20480
_ROWS = _PPAD // _LANES


def _odm_loss_kernel(targets_ref, arm_loc_ref, arm_conf_ref, odm_loc_ref,
                     odm_conf_ref, priors_ref, out_l_ref, out_c_ref, acc_ref):
    i = pl.program_id(0)

    @pl.when(i == 0)
    def _init():
        acc_ref[0] = 0.0
        acc_ref[1] = 0.0
        acc_ref[2] = 0.0

    al = arm_loc_ref[0]
    ac = arm_conf_ref[0]
    ol = odm_loc_ref[0]
    oc = odm_conf_ref[0]
    pr = priors_ref[...]

    c0, c1 = ac[0], ac[1]
    cm = jnp.maximum(c0, c1)
    e0 = jnp.exp(c0 - cm)
    e1 = jnp.exp(c1 - cm)
    reserve = e1 / (e0 + e1) > _POS_PRIOR_THRESHOLD

    pcx, pcy, pw, ph = pr[0], pr[1], pr[2], pr[3]
    cx = pcx + al[0] * 0.1 * pw
    cy = pcy + al[1] * 0.1 * ph
    w = pw * jnp.exp(al[2] * 0.2)
    h = ph * jnp.exp(al[3] * 0.2)
    x1 = cx - w / 2.0
    y1 = cy - h / 2.0
    x2 = cx + w / 2.0
    y2 = cy + h / 2.0
    area_b = (x2 - x1) * (y2 - y1)

    flat = (jax.lax.broadcasted_iota(jnp.int32, (_ROWS, _LANES), 0) * _LANES
            + jax.lax.broadcasted_iota(jnp.int32, (_ROWS, _LANES), 1))

    ious = []
    bpi = []
    for t in range(_T):
        tx1 = targets_ref[0, t, 0]
        ty1 = targets_ref[0, t, 1]
        tx2 = targets_ref[0, t, 2]
        ty2 = targets_ref[0, t, 3]
        area_a = (tx2 - tx1) * (ty2 - ty1)
        ltx = jnp.maximum(tx1, x1)
        lty = jnp.maximum(ty1, y1)
        rbx = jnp.minimum(tx2, x2)
        rby = jnp.minimum(ty2, y2)
        iw = jnp.clip(rbx - ltx, 0.0, None)
        ih = jnp.clip(rby - lty, 0.0, None)
        inter = iw * ih
        iou = inter / (area_a + area_b - inter)
        ious.append(iou)
        ovr = jnp.where(reserve, iou, -1.0)
        mx = jnp.max(ovr)
        bpi.append(jnp.min(jnp.where(ovr == mx, flat, jnp.int32(2**30))))

    bto = ious[0]
    bti = jnp.zeros_like(flat)
    for t in range(1, _T):
        upd = ious[t] > bto
        bto = jnp.where(upd, ious[t], bto)
        bti = jnp.where(upd, t, bti)
    for t in range(_T):
        hit = flat == bpi[t]
        bto = jnp.where(hit, 2.0, bto)
        bti = jnp.where(hit, t, bti)

    zero = jnp.zeros_like(bto)
    mx1 = zero
    my1 = zero
    mx2 = zero
    my2 = zero
    lab = zero
    for t in range(_T):
        sel = bti == t
        mx1 = jnp.where(sel, targets_ref[0, t, 0], mx1)
        my1 = jnp.where(sel, targets_ref[0, t, 1], my1)
        mx2 = jnp.where(sel, targets_ref[0, t, 2], mx2)
        my2 = jnp.where(sel, targets_ref[0, t, 3], my2)
        lab = jnp.where(sel, targets_ref[0, t, 4], lab)

    conf = (lab + 1.0).astype(jnp.int32)
    conf = jnp.where(bto < _OVERLAP_THRESH, 0, conf)
    conf = jnp.where(reserve, conf, -1)
    pos = conf > 0
    ign = conf == -1

    g_cx = ((mx1 + mx2) / 2.0 - cx) / (0.1 * w)
    g_cy = ((my1 + my2) / 2.0 - cy) / (0.1 * h)
    g_w = jnp.log(jnp.clip((mx2 - mx1) / w, 1e-8, None)) / 0.2
    g_h = jnp.log(jnp.clip((my2 - my1) / h, 1e-8, None)) / 0.2

    loss_l = zero
    for d, g in ((ol[0], g_cx), (ol[1], g_cy), (ol[2], g_w), (ol[3], g_h)):
        diff = d - g
        ad = jnp.abs(diff)
        sl1 = jnp.where(ad < 1.0, 0.5 * diff * diff, ad - 0.5)
        loss_l = loss_l + jnp.where(pos, sl1, 0.0)
    loss_l_i = jnp.sum(loss_l)

    m2 = oc[0]
    for c in range(1, _NUM_CLASSES):
        m2 = jnp.maximum(m2, oc[c])
    s = jnp.zeros_like(m2)
    for c in range(_NUM_CLASSES):
        s = s + jnp.exp(oc[c] - m2)
    lse = m2 + jnp.log(s)
    conf0 = jnp.where(ign, 0, conf)
    gathered = zero
    for c in range(_NUM_CLASSES):
        gathered = jnp.where(conf0 == c, oc[c], gathered)
    lmg = lse - gathered

    num_pos = jnp.sum(pos.astype(jnp.int32))
    loss_c_pos = jnp.sum(jnp.where(pos, lmg, 0.0))
    rank = jnp.where(pos | ign, 0.0, lmg)

    k = jnp.minimum(_NEG_POS_RATIO * num_pos, _P - 1)
    bits = jax.lax.bitcast_convert_type(rank, jnp.int32)
    kth = jnp.int32(0)
    for b in range(30, -1, -1):
        cand = kth | jnp.int32(1 << b)
        cnt = jnp.sum((bits >= cand).astype(jnp.int32))
        kth = jnp.where(cnt >= k, cand, kth)
    cnt_gt = jnp.sum((bits > kth).astype(jnp.int32))
    sum_gt = jnp.sum(jnp.where(bits > kth, rank, 0.0))
    kth_val = jax.lax.bitcast_convert_type(kth, jnp.float32)
    topk = jnp.where(
        k > 0, sum_gt + (k - cnt_gt).astype(jnp.float32) * kth_val, 0.0)

    acc_ref[0] = acc_ref[0] + loss_l_i
    acc_ref[1] = acc_ref[1] + loss_c_pos + topk
    acc_ref[2] = acc_ref[2] + num_pos.astype(jnp.float32)

    @pl.when(i == _N - 1)
    def _fin():
        total = acc_ref[2]
        out_l_ref[0, 0] = acc_ref[0] / total
        out_c_ref[0, 0] = acc_ref[1] / total


def _prep(x, pad_row):
    n, _, c = x.shape
    tail = jnp.broadcast_to(jnp.asarray(pad_row, x.dtype), (n, _PPAD - _P, c))
    xp = jnp.concatenate([x, tail], axis=1)
    return xp.transpose(0, 2, 1).reshape(n, c, _ROWS, _LANES)


def kernel(arm_loc, arm_conf, odm_loc, odm_conf, priors, targets):
    al = _prep(arm_loc, [0.0, 0.0, 0.0, 0.0])
    ac = _prep(arm_conf, [40.0, -40.0])
    ol = _prep(odm_loc, [0.0, 0.0, 0.0, 0.0])
    oc = _prep(odm_conf, [0.0] * _NUM_CLASSES)
    pr_tail = jnp.broadcast_to(
        jnp.asarray([0.5, 0.5, 1.0, 1.0], priors.dtype), (_PPAD - _P, 4))
    pr = jnp.concatenate([priors, pr_tail], axis=0)
    pr = pr.transpose(1, 0).reshape(4, _ROWS, _LANES)

    grid = (_N,)
    out_l, out_c = pl.pallas_call(
        _odm_loss_kernel,
        grid=grid,
        in_specs=[
            pl.BlockSpec((1, _T, 5), lambda i: (i, 0, 0),
                         memory_space=pltpu.SMEM),
            pl.BlockSpec((1, 4, _ROWS, _LANES), lambda i: (i, 0, 0, 0)),
            pl.BlockSpec((1, 2, _ROWS, _LANES), lambda i: (i, 0, 0, 0)),
            pl.BlockSpec((1, 4, _ROWS, _LANES), lambda i: (i, 0, 0, 0)),
            pl.BlockSpec((1, _NUM_CLASSES, _ROWS, _LANES),
                         lambda i: (i, 0, 0, 0)),
            pl.BlockSpec((4, _ROWS, _LANES), lambda i: (0, 0, 0)),
        ],
        out_specs=[
            pl.BlockSpec((1, 1), lambda i: (0, 0), memory_space=pltpu.SMEM),
            pl.BlockSpec((1, 1), lambda i: (0, 0), memory_space=pltpu.SMEM),
        ],
        out_shape=[
            jax.ShapeDtypeStruct((1, 1), jnp.float32),
            jax.ShapeDtypeStruct((1, 1), jnp.float32),
        ],
        scratch_shapes=[pltpu.SMEM((4,), jnp.float32)],
        compiler_params=pltpu.CompilerParams(
            dimension_semantics=("arbitrary",)),
    )(targets, al, ac, ol, oc, pr)
    return out_l.reshape(()), out_c.reshape(())
